# baseline (speedup 1.0000x reference)
import functools

import jax
import jax.numpy as jnp
from jax.experimental import pallas as pl
from jax.experimental.pallas import tpu as pltpu


def _fused_row_block(scal_ref, adj_ref, x_ref, h0_ref, w_ref, o_ref, *, bm):
    theta = scal_ref[0]
    alpha = scal_ref[1]
    i = pl.program_id(0)
    hi = jnp.dot(adj_ref[...], x_ref[...], preferred_element_type=jnp.float32)
    support = (1.0 - alpha) * hi + alpha * h0_ref[...]
    xb = x_ref[pl.ds(i * bm, bm), :]
    o_ref[...] = (
        theta * jnp.dot(support, w_ref[...], preferred_element_type=jnp.float32)
        + (1.0 - theta) * support
        + xb
    )


def kernel(input, adj, h0, lamda, alpha, l, weight):
    n, d = input.shape
    theta = jnp.minimum(1.0, jnp.log(lamda / l + 1.0))
    scal = jnp.stack([theta.astype(jnp.float32), alpha.astype(jnp.float32)])

    bm = 400 if n % 400 == 0 else n

    return pl.pallas_call(
        functools.partial(_fused_row_block, bm=bm),
        grid=(n // bm,),
        in_specs=[
            pl.BlockSpec(memory_space=pltpu.SMEM),
            pl.BlockSpec((bm, n), lambda i: (i, 0)),
            pl.BlockSpec((n, d), lambda i: (0, 0)),
            pl.BlockSpec((bm, d), lambda i: (i, 0)),
            pl.BlockSpec((d, d), lambda i: (0, 0)),
        ],
        out_specs=pl.BlockSpec((bm, d), lambda i: (i, 0)),
        out_shape=jax.ShapeDtypeStruct((n, d), jnp.float32),
        compiler_params=pltpu.CompilerParams(
            dimension_semantics=("arbitrary",),
        ),
    )(scal, adj, input, h0, weight)


# speedup vs baseline: 1.0094x; 1.0094x over previous
import functools

import jax
import jax.numpy as jnp
from jax.experimental import pallas as pl
from jax.experimental.pallas import tpu as pltpu


def _fused_row_block(lam_ref, al_ref, l_ref, adj_ref, x_ref, h0_ref, w_ref, o_ref, *, bm):
    theta = jnp.minimum(1.0, jnp.log(lam_ref[0] / l_ref[0] + 1.0))
    alpha = al_ref[0]
    i = pl.program_id(0)
    hi = jnp.dot(adj_ref[...], x_ref[...], preferred_element_type=jnp.float32)
    support = (1.0 - alpha) * hi + alpha * h0_ref[...]
    xb = x_ref[pl.ds(i * bm, bm), :]
    o_ref[...] = (
        theta * jnp.dot(support, w_ref[...], preferred_element_type=jnp.float32)
        + (1.0 - theta) * support
        + xb
    )


def kernel(input, adj, h0, lamda, alpha, l, weight):
    n, d = input.shape
    lam = jnp.reshape(lamda.astype(jnp.float32), (1,))
    al = jnp.reshape(alpha.astype(jnp.float32), (1,))
    lv = jnp.reshape(jnp.asarray(l).astype(jnp.float32), (1,))

    bm = 200 if n % 200 == 0 else n

    return pl.pallas_call(
        functools.partial(_fused_row_block, bm=bm),
        grid=(n // bm,),
        in_specs=[
            pl.BlockSpec(memory_space=pltpu.SMEM),
            pl.BlockSpec(memory_space=pltpu.SMEM),
            pl.BlockSpec(memory_space=pltpu.SMEM),
            pl.BlockSpec((bm, n), lambda i: (i, 0)),
            pl.BlockSpec((n, d), lambda i: (0, 0)),
            pl.BlockSpec((bm, d), lambda i: (i, 0)),
            pl.BlockSpec((d, d), lambda i: (0, 0)),
        ],
        out_specs=pl.BlockSpec((bm, d), lambda i: (i, 0)),
        out_shape=jax.ShapeDtypeStruct((n, d), jnp.float32),
        compiler_params=pltpu.CompilerParams(
            dimension_semantics=("arbitrary",),
        ),
    )(lam, al, lv, adj, input, h0, weight)
